# baseline (speedup 1.0000x reference)
import jax
import jax.numpy as jnp
from jax import lax
from jax.experimental import pallas as pl
from jax.experimental.pallas import tpu as pltpu
from jax.experimental.pallas import tpu_sc as plsc

NODES_NUM = 1000
EMB_SIZE = 128
BATCH = 4096
SEQ = 50

NC, NS = 2, 16
NW = NC * NS
TOTAL = BATCH * SEQ
PER_W = TOTAL // NW
CHUNK = 128
NCHUNK = PER_W // CHUNK
OUT_W = EMB_SIZE + 2
MPC = 4
NPACK = BATCH // MPC
PACK_PER_W = NPACK // NW
PROWS = MPC * SEQ
PMIN = 256


def _sc_gather_body(idx_hbm, table_hbm, emb_hbm, idx_v, rows_v, sem):
    wid = lax.axis_index("s") * NC + lax.axis_index("c")

    @pl.loop(0, NCHUNK)
    def _chunk(g):
        base = wid * PER_W + g * CHUNK
        pltpu.sync_copy(idx_hbm.at[pl.ds(base, CHUNK)], idx_v)
        pltpu.async_copy(table_hbm.at[idx_v], rows_v, sem).wait()
        pltpu.sync_copy(rows_v, emb_hbm.at[pl.ds(base, CHUNK)])


def _sc_pack_body(emb_hbm, ch_hbm, rad_hbm, out_hbm,
                  ch_v, rad_v, rows_v, out_v):
    wid = lax.axis_index("s") * NC + lax.axis_index("c")
    lanes = lax.iota(jnp.int32, 16)
    czero = jnp.zeros((16,), jnp.int32)
    c129 = jnp.full((16,), OUT_W - 1, jnp.int32)

    @pl.loop(0, PACK_PER_W)
    def _chunk(m):
        cid = wid * PACK_PER_W + m
        pltpu.sync_copy(emb_hbm.at[pl.ds(cid * PROWS, PROWS)], rows_v)
        pltpu.sync_copy(ch_hbm.at[pl.ds(cid, 1)], ch_v)
        pltpu.sync_copy(rad_hbm.at[pl.ds(cid, 1)], rad_v)
        for mb in range(MPC):
            lane0 = mb * SEQ
            mbv = jnp.full((16,), mb, jnp.int32)
            for i in range(lane0 // 16, (lane0 + SEQ - 1) // 16 + 1):
                rows = lanes + 16 * i - lane0
                rcl = jnp.clip(rows, 0, SEQ - 1)
                inb = (rows >= 0) & (rows < SEQ)
                full = bool((16 * i >= lane0) and (16 * i + 16 <= lane0 + SEQ))
                mask = None if full else inb
                ch = ch_v[0, pl.ds(i * 16, 16)]
                rd = rad_v[0, pl.ds(i * 16, 16)]
                plsc.store_scatter(out_v, [mbv, rcl, czero], ch, mask=mask)
                plsc.store_scatter(out_v, [mbv, rcl, c129], rd, mask=mask)

            @pl.loop(0, SEQ)
            def _row(r):
                rvec = czero + r
                gvec = czero + (lane0 + r)
                for j in range(EMB_SIZE // 16):
                    v = plsc.load_gather(rows_v, [gvec, lanes + j * 16])
                    plsc.store_scatter(
                        out_v, [mbv, rvec, lanes + (j * 16 + 1)], v)

        pltpu.sync_copy(out_v, out_hbm.at[pl.ds(cid * MPC, MPC)])


@jax.jit
def kernel(inputs, emb_table):
    idx = inputs[..., 0].astype(jnp.int32).reshape(TOTAL)
    pad = ((0, 0), (0, PMIN - PROWS))
    ch = jnp.pad(inputs[..., 1].reshape(NPACK, PROWS), pad)
    rad = jnp.pad(inputs[..., 2].reshape(NPACK, PROWS), pad)
    mesh = plsc.VectorSubcoreMesh(core_axis_name="c", subcore_axis_name="s")

    emb = pl.kernel(
        _sc_gather_body,
        out_type=jax.ShapeDtypeStruct((TOTAL, EMB_SIZE), jnp.float32),
        mesh=mesh,
        scratch_types=[
            pltpu.VMEM((CHUNK,), jnp.int32),
            pltpu.VMEM((CHUNK, EMB_SIZE), jnp.float32),
            pltpu.SemaphoreType.DMA,
        ],
        compiler_params=pltpu.CompilerParams(
            use_tc_tiling_on_sc=False, needs_layout_passes=False),
    )(idx, emb_table)

    return pl.kernel(
        _sc_pack_body,
        out_type=jax.ShapeDtypeStruct((BATCH, SEQ, OUT_W), jnp.float32),
        mesh=mesh,
        scratch_types=[
            pltpu.VMEM((1, PMIN), jnp.float32),
            pltpu.VMEM((1, PMIN), jnp.float32),
            pltpu.VMEM((PROWS, EMB_SIZE), jnp.float32),
            pltpu.VMEM((MPC, SEQ, OUT_W), jnp.float32),
        ],
        compiler_params=pltpu.CompilerParams(
            use_tc_tiling_on_sc=True, needs_layout_passes=False),
    )(emb, ch, rad)


# speedup vs baseline: 1.0026x; 1.0026x over previous
import jax
import jax.numpy as jnp
from jax import lax
from jax.experimental import pallas as pl
from jax.experimental.pallas import tpu as pltpu
from jax.experimental.pallas import tpu_sc as plsc

NODES_NUM = 1000
EMB_SIZE = 128
BATCH = 4096
SEQ = 50

NC, NS = 2, 16
NW = NC * NS
TOTAL = BATCH * SEQ
PER_W = TOTAL // NW
CHUNK = 128
NCHUNK = PER_W // CHUNK
OUT_W = EMB_SIZE + 2
MPC = 4
NPACK = BATCH // MPC
PACK_PER_W = NPACK // NW
PROWS = MPC * SEQ
PMIN = 256


def _sc_gather_body(idx_hbm, table_hbm, emb_hbm, idx_v, rows_v, sem):
    wid = lax.axis_index("s") * NC + lax.axis_index("c")

    @pl.loop(0, NCHUNK)
    def _chunk(g):
        base = wid * PER_W + g * CHUNK
        pltpu.sync_copy(idx_hbm.at[pl.ds(base, CHUNK)], idx_v)
        pltpu.async_copy(table_hbm.at[idx_v], rows_v, sem).wait()
        pltpu.sync_copy(rows_v, emb_hbm.at[pl.ds(base, CHUNK)])


def _sc_pack_body(emb_hbm, ch_hbm, rad_hbm, out_hbm,
                  ch_v, rad_v, rows_v, out_v):
    wid = lax.axis_index("s") * NC + lax.axis_index("c")
    lanes = lax.iota(jnp.int32, 16)
    czero = jnp.zeros((16,), jnp.int32)
    c129 = jnp.full((16,), OUT_W - 1, jnp.int32)

    @pl.loop(0, PACK_PER_W)
    def _chunk(m):
        cid = wid * PACK_PER_W + m
        pltpu.sync_copy(emb_hbm.at[pl.ds(cid * PROWS, PROWS)], rows_v)
        pltpu.sync_copy(ch_hbm.at[pl.ds(cid, 1)], ch_v)
        pltpu.sync_copy(rad_hbm.at[pl.ds(cid, 1)], rad_v)
        for mb in range(MPC):
            lane0 = mb * SEQ
            for i in range(lane0 // 16, (lane0 + SEQ - 1) // 16 + 1):
                rows = lanes + 16 * i - lane0
                rcl = jnp.clip(rows, 0, SEQ - 1)
                inb = (rows >= 0) & (rows < SEQ)
                full = bool((16 * i >= lane0) and (16 * i + 16 <= lane0 + SEQ))
                mask = None if full else inb
                ch = ch_v[0, pl.ds(i * 16, 16)]
                rd = rad_v[0, pl.ds(i * 16, 16)]
                plsc.store_scatter(out_v, [rcl, czero], ch, mask=mask)
                plsc.store_scatter(out_v, [rcl, c129], rd, mask=mask)

            @pl.loop(0, SEQ)
            def _row(r):
                rvec = czero + r
                gvec = czero + (lane0 + r)
                for j in range(EMB_SIZE // 16):
                    v = plsc.load_gather(rows_v, [gvec, lanes + j * 16])
                    plsc.store_scatter(
                        out_v, [rvec, lanes + (j * 16 + 1)], v)

            pltpu.sync_copy(out_v, out_hbm.at[cid * MPC + mb])


@jax.jit
def kernel(inputs, emb_table):
    idx = inputs[..., 0].astype(jnp.int32).reshape(TOTAL)
    pad = ((0, 0), (0, PMIN - PROWS))
    ch = jnp.pad(inputs[..., 1].reshape(NPACK, PROWS), pad)
    rad = jnp.pad(inputs[..., 2].reshape(NPACK, PROWS), pad)
    mesh = plsc.VectorSubcoreMesh(core_axis_name="c", subcore_axis_name="s")

    emb = pl.kernel(
        _sc_gather_body,
        out_type=jax.ShapeDtypeStruct((TOTAL, EMB_SIZE), jnp.float32),
        mesh=mesh,
        scratch_types=[
            pltpu.VMEM((CHUNK,), jnp.int32),
            pltpu.VMEM((CHUNK, EMB_SIZE), jnp.float32),
            pltpu.SemaphoreType.DMA,
        ],
        compiler_params=pltpu.CompilerParams(
            use_tc_tiling_on_sc=False, needs_layout_passes=False),
    )(idx, emb_table)

    return pl.kernel(
        _sc_pack_body,
        out_type=jax.ShapeDtypeStruct((BATCH, SEQ, OUT_W), jnp.float32),
        mesh=mesh,
        scratch_types=[
            pltpu.VMEM((1, PMIN), jnp.float32),
            pltpu.VMEM((1, PMIN), jnp.float32),
            pltpu.VMEM((PROWS, EMB_SIZE), jnp.float32),
            pltpu.VMEM((SEQ, OUT_W), jnp.float32),
        ],
        compiler_params=pltpu.CompilerParams(
            use_tc_tiling_on_sc=True, needs_layout_passes=False),
    )(emb, ch, rad)
